# Initial kernel scaffold; baseline (speedup 1.0000x reference)
import functools

import jax
import jax.numpy as jnp
from jax import lax
from jax.experimental import pallas as pl
from jax.experimental.pallas import tpu as pltpu
from jax.experimental.pallas import tpu_sc as plsc

N = 10000
E = 320000
D = 128

C = 128
NCHUNKS = E // C
NW = 32
ROWS_PER_SUB = N // 16


def _node_mlp(x, W1, b1, W2, b2):
    BN = 2000

    def body(x_ref, w1_ref, b1_ref, w2_ref, b2_ref, o_ref):
        h = jnp.tanh(
            jnp.dot(x_ref[...], w1_ref[...], preferred_element_type=jnp.float32)
            + b1_ref[...]
        )
        o_ref[...] = (
            jnp.dot(h, w2_ref[...], preferred_element_type=jnp.float32)
            + b2_ref[...]
        )

    wspec = pl.BlockSpec((D, D), lambda i: (0, 0))
    bspec = pl.BlockSpec((1, D), lambda i: (0, 0))
    return pl.pallas_call(
        body,
        grid=(N // BN,),
        in_specs=[
            pl.BlockSpec((BN, D), lambda i: (i, 0)),
            wspec, bspec, wspec, bspec,
        ],
        out_specs=pl.BlockSpec((BN, D), lambda i: (i, 0)),
        out_shape=jax.ShapeDtypeStruct((N, D), jnp.float32),
    )(x, W1, b1, W2, b2)


def _edge_mlps(x, W1, b1, W2, b2, W3, b3, W4, b4):
    BE = 2000

    def body(x_ref, w1_ref, b1_ref, w2_ref, b2_ref,
             w3_ref, b3_ref, w4_ref, b4_ref, o_ref):
        h1 = jnp.tanh(
            jnp.dot(x_ref[...], w1_ref[...], preferred_element_type=jnp.float32)
            + b1_ref[...]
        )
        p = (
            jnp.dot(h1, w2_ref[...], preferred_element_type=jnp.float32)
            + b2_ref[...]
        )
        h2 = jnp.tanh(
            jnp.dot(p, w3_ref[...], preferred_element_type=jnp.float32)
            + b3_ref[...]
        )
        o_ref[...] = (
            jnp.dot(h2, w4_ref[...], preferred_element_type=jnp.float32)
            + b4_ref[...]
        )

    wspec = pl.BlockSpec((D, D), lambda i: (0, 0))
    bspec = pl.BlockSpec((1, D), lambda i: (0, 0))
    return pl.pallas_call(
        body,
        grid=(E // BE,),
        in_specs=[
            pl.BlockSpec((BE, D), lambda i: (i, 0)),
            wspec, bspec, wspec, bspec, wspec, bspec, wspec, bspec,
        ],
        out_specs=pl.BlockSpec((BE, D), lambda i: (i, 0)),
        out_shape=jax.ShapeDtypeStruct((E, D), jnp.float32),
    )(x, W1, b1, W2, b2, W3, b3, W4, b4)


def _add_partials(parts):
    BN = 2000

    def body(p_ref, o_ref):
        o_ref[...] = p_ref[0] + p_ref[1]

    return pl.pallas_call(
        body,
        grid=(N // BN,),
        in_specs=[pl.BlockSpec((2, BN, D), lambda i: (0, i, 0))],
        out_specs=pl.BlockSpec((BN, D), lambda i: (i, 0)),
        out_shape=jax.ShapeDtypeStruct((N, D), jnp.float32),
    )(parts)


_MESH = plsc.VectorSubcoreMesh(core_axis_name="c", subcore_axis_name="s")


@functools.partial(
    pl.kernel,
    mesh=_MESH,
    out_type=jax.ShapeDtypeStruct((E, D), jnp.float32),
    scratch_types=[
        pltpu.VMEM((C,), jnp.int32),
        pltpu.VMEM((C,), jnp.int32),
        pltpu.VMEM((C, D), jnp.float32),
        pltpu.VMEM((C, D), jnp.float32),
        pltpu.VMEM((C, D), jnp.float32),
        pltpu.SemaphoreType.DMA,
        pltpu.SemaphoreType.DMA,
    ],
)
def _gather_sum(idx_i_hbm, idx_j_hbm, pp1_hbm, basis_hbm, inter_hbm,
                idxi_v, idxj_v, rows_i, rows_j, acc, sem_i, sem_j):
    c = lax.axis_index("c")
    s = lax.axis_index("s")
    w = s * 2 + c
    nchunks = 78 + (w < 4).astype(jnp.int32)

    def chunk(t, carry):
        g = w + t * NW
        base = g * C
        pltpu.sync_copy(idx_i_hbm.at[pl.ds(base, C)], idxi_v)
        pltpu.sync_copy(idx_j_hbm.at[pl.ds(base, C)], idxj_v)
        cp_i = pltpu.async_copy(pp1_hbm.at[idxi_v], rows_i, sem_i)
        cp_j = pltpu.async_copy(pp1_hbm.at[idxj_v], rows_j, sem_j)
        pltpu.sync_copy(basis_hbm.at[pl.ds(base, C)], acc)
        cp_i.wait()
        cp_j.wait()

        def row(r, carry2):
            for k in range(D // 16):
                sl = pl.ds(k * 16, 16)
                acc[r, sl] += rows_i[r, sl] + rows_j[r, sl]
            return carry2

        lax.fori_loop(0, C, row, 0)
        pltpu.sync_copy(acc, inter_hbm.at[pl.ds(base, C)])
        return carry

    lax.fori_loop(0, nchunks, chunk, 0)


@functools.partial(
    pl.kernel,
    mesh=_MESH,
    out_type=jax.ShapeDtypeStruct((2, N, D), jnp.float32),
    scratch_types=[
        pltpu.VMEM((C,), jnp.int32),
        pltpu.VMEM((C, D), jnp.float32),
        pltpu.VMEM_SHARED((N, D), jnp.float32),
    ],
)
def _scatter_add(idx_i_hbm, ii1_hbm, zeros_hbm, out_hbm,
                 idx_v, rows_v, acc_shared):
    c = lax.axis_index("c")
    s = lax.axis_index("s")
    r0 = s * ROWS_PER_SUB
    pltpu.sync_copy(zeros_hbm.at[pl.ds(r0, ROWS_PER_SUB)],
                    acc_shared.at[pl.ds(r0, ROWS_PER_SUB)])
    plsc.subcore_barrier()

    half = NCHUNKS // 2
    nchunks = 78 + (s < 2).astype(jnp.int32)

    def chunk(t, carry):
        g = c * half + s + t * 16
        base = g * C
        pltpu.sync_copy(idx_i_hbm.at[pl.ds(base, C)], idx_v)
        pltpu.sync_copy(ii1_hbm.at[pl.ds(base, C)], rows_v)
        pltpu.sync_copy(rows_v, acc_shared.at[idx_v], add=True)
        return carry

    lax.fori_loop(0, nchunks, chunk, 0)
    plsc.subcore_barrier()
    pltpu.sync_copy(acc_shared.at[pl.ds(r0, ROWS_PER_SUB)],
                    out_hbm.at[c, pl.ds(r0, ROWS_PER_SUB)])


def kernel(idx_i, idx_j, p1, basis,
           pp_W1, pp_b1, pp_W2, pp_b2,
           pi_W1, pi_b1, pi_W2, pi_b2,
           ii_W1, ii_b1, ii_W2, ii_b2):
    idx_i = idx_i.astype(jnp.int32)
    idx_j = idx_j.astype(jnp.int32)
    b = lambda v: v.reshape(1, D)

    pp1 = _node_mlp(p1, pp_W1, b(pp_b1), pp_W2, b(pp_b2))
    inter = _gather_sum(idx_i, idx_j, pp1, basis)
    ii1 = _edge_mlps(inter,
                     pi_W1, b(pi_b1), pi_W2, b(pi_b2),
                     ii_W1, b(ii_b1), ii_W2, b(ii_b2))
    zeros = jnp.zeros((N, D), jnp.float32)
    parts = _scatter_add(idx_i, ii1, zeros)
    return _add_partials(parts)


# speedup vs baseline: 3.1779x; 3.1779x over previous
import functools

import jax
import jax.numpy as jnp
from jax import lax
from jax.experimental import pallas as pl
from jax.experimental.pallas import tpu as pltpu
from jax.experimental.pallas import tpu_sc as plsc

N = 10000
E = 320000
D = 128

C = 128
NCHUNKS = E // C
NW = 32
ROWS_PER_SUB = 624
ROWS_LAST = N - 15 * ROWS_PER_SUB


def _node_mlp(x, W1, b1, W2, b2):
    BN = 2000

    def body(x_ref, w1_ref, b1_ref, w2_ref, b2_ref, o_ref):
        h = jnp.tanh(
            jnp.dot(x_ref[...], w1_ref[...], preferred_element_type=jnp.float32)
            + b1_ref[...]
        )
        o_ref[...] = (
            jnp.dot(h, w2_ref[...], preferred_element_type=jnp.float32)
            + b2_ref[...]
        )

    wspec = pl.BlockSpec((D, D), lambda i: (0, 0))
    bspec = pl.BlockSpec((1, D), lambda i: (0, 0))
    return pl.pallas_call(
        body,
        grid=(N // BN,),
        in_specs=[
            pl.BlockSpec((BN, D), lambda i: (i, 0)),
            wspec, bspec, wspec, bspec,
        ],
        out_specs=pl.BlockSpec((BN, D), lambda i: (i, 0)),
        out_shape=jax.ShapeDtypeStruct((N, D), jnp.float32),
    )(x, W1, b1, W2, b2)


def _edge_mlps(x, W1, b1, W2, b2, W3, b3, W4, b4):
    BE = 2000

    def body(x_ref, w1_ref, b1_ref, w2_ref, b2_ref,
             w3_ref, b3_ref, w4_ref, b4_ref, o_ref):
        h1 = jnp.tanh(
            jnp.dot(x_ref[...], w1_ref[...], preferred_element_type=jnp.float32)
            + b1_ref[...]
        )
        p = (
            jnp.dot(h1, w2_ref[...], preferred_element_type=jnp.float32)
            + b2_ref[...]
        )
        h2 = jnp.tanh(
            jnp.dot(p, w3_ref[...], preferred_element_type=jnp.float32)
            + b3_ref[...]
        )
        o_ref[...] = (
            jnp.dot(h2, w4_ref[...], preferred_element_type=jnp.float32)
            + b4_ref[...]
        )

    wspec = pl.BlockSpec((D, D), lambda i: (0, 0))
    bspec = pl.BlockSpec((1, D), lambda i: (0, 0))
    return pl.pallas_call(
        body,
        grid=(E // BE,),
        in_specs=[
            pl.BlockSpec((BE, D), lambda i: (i, 0)),
            wspec, bspec, wspec, bspec, wspec, bspec, wspec, bspec,
        ],
        out_specs=pl.BlockSpec((BE, D), lambda i: (i, 0)),
        out_shape=jax.ShapeDtypeStruct((E, D), jnp.float32),
    )(x, W1, b1, W2, b2, W3, b3, W4, b4)


def _add_partials(parts):
    BN = 2000

    def body(p_ref, o_ref):
        o_ref[...] = p_ref[0] + p_ref[1]

    return pl.pallas_call(
        body,
        grid=(N // BN,),
        in_specs=[pl.BlockSpec((2, BN, D), lambda i: (0, i, 0))],
        out_specs=pl.BlockSpec((BN, D), lambda i: (i, 0)),
        out_shape=jax.ShapeDtypeStruct((N, D), jnp.float32),
    )(parts)


_MESH = plsc.VectorSubcoreMesh(core_axis_name="c", subcore_axis_name="s")


@functools.partial(
    pl.kernel,
    mesh=_MESH,
    out_type=jax.ShapeDtypeStruct((E, D), jnp.float32),
    scratch_types=[
        pltpu.VMEM((C,), jnp.int32),
        pltpu.VMEM((C,), jnp.int32),
        pltpu.VMEM((C, D), jnp.float32),
        pltpu.VMEM((C, D), jnp.float32),
        pltpu.VMEM((C, D), jnp.float32),
        pltpu.SemaphoreType.DMA,
        pltpu.SemaphoreType.DMA,
    ],
)
def _gather_sum(idx_i_hbm, idx_j_hbm, pp1_hbm, basis_hbm, inter_hbm,
                idxi_v, idxj_v, rows_i, rows_j, acc, sem_i, sem_j):
    c = lax.axis_index("c")
    s = lax.axis_index("s")
    w = s * 2 + c
    nchunks = 78 + (w < 4).astype(jnp.int32)

    def chunk(t, carry):
        g = w + t * NW
        base = pl.multiple_of(g * C, C)
        pltpu.sync_copy(idx_i_hbm.at[pl.ds(base, C)], idxi_v)
        pltpu.sync_copy(idx_j_hbm.at[pl.ds(base, C)], idxj_v)
        cp_i = pltpu.async_copy(pp1_hbm.at[idxi_v], rows_i, sem_i)
        cp_j = pltpu.async_copy(pp1_hbm.at[idxj_v], rows_j, sem_j)
        pltpu.sync_copy(basis_hbm.at[pl.ds(base, C)], acc)
        cp_i.wait()
        cp_j.wait()

        def row(r, carry2):
            for k in range(D // 16):
                sl = pl.ds(k * 16, 16)
                acc[r, sl] += rows_i[r, sl] + rows_j[r, sl]
            return carry2

        lax.fori_loop(0, C, row, 0)
        pltpu.sync_copy(acc, inter_hbm.at[pl.ds(base, C)])
        return carry

    lax.fori_loop(0, nchunks, chunk, 0)


@functools.partial(
    pl.kernel,
    mesh=_MESH,
    out_type=jax.ShapeDtypeStruct((2, N, D), jnp.float32),
    scratch_types=[
        pltpu.VMEM((C,), jnp.int32),
        pltpu.VMEM((C, D), jnp.float32),
        pltpu.VMEM_SHARED((N, D), jnp.float32),
    ],
)
def _scatter_add(idx_i_hbm, ii1_hbm, zeros_hbm, out_hbm,
                 idx_v, rows_v, acc_shared):
    c = lax.axis_index("c")
    s = lax.axis_index("s")
    r0 = pl.multiple_of(s * ROWS_PER_SUB, 8)
    pltpu.sync_copy(zeros_hbm.at[pl.ds(r0, ROWS_PER_SUB)],
                    acc_shared.at[pl.ds(r0, ROWS_PER_SUB)])

    @pl.when(s == 15)
    def _():
        tail = 15 * ROWS_PER_SUB + ROWS_PER_SUB
        pltpu.sync_copy(zeros_hbm.at[pl.ds(tail, ROWS_LAST - ROWS_PER_SUB)],
                        acc_shared.at[pl.ds(tail, ROWS_LAST - ROWS_PER_SUB)])

    plsc.subcore_barrier()

    half = NCHUNKS // 2
    nchunks = 78 + (s < 2).astype(jnp.int32)

    def chunk(t, carry):
        g = c * half + s + t * 16
        base = pl.multiple_of(g * C, C)
        pltpu.sync_copy(idx_i_hbm.at[pl.ds(base, C)], idx_v)
        pltpu.sync_copy(ii1_hbm.at[pl.ds(base, C)], rows_v)
        pltpu.sync_copy(rows_v, acc_shared.at[idx_v], add=True)
        return carry

    lax.fori_loop(0, nchunks, chunk, 0)
    plsc.subcore_barrier()
    pltpu.sync_copy(acc_shared.at[pl.ds(r0, ROWS_PER_SUB)],
                    out_hbm.at[c, pl.ds(r0, ROWS_PER_SUB)])

    @pl.when(s == 15)
    def _():
        tail = 15 * ROWS_PER_SUB + ROWS_PER_SUB
        pltpu.sync_copy(acc_shared.at[pl.ds(tail, ROWS_LAST - ROWS_PER_SUB)],
                        out_hbm.at[c, pl.ds(tail, ROWS_LAST - ROWS_PER_SUB)])


def kernel(idx_i, idx_j, p1, basis,
           pp_W1, pp_b1, pp_W2, pp_b2,
           pi_W1, pi_b1, pi_W2, pi_b2,
           ii_W1, ii_b1, ii_W2, ii_b2):
    idx_i = idx_i.astype(jnp.int32)
    idx_j = idx_j.astype(jnp.int32)
    b = lambda v: v.reshape(1, D)

    pp1 = _node_mlp(p1, pp_W1, b(pp_b1), pp_W2, b(pp_b2))
    inter = _gather_sum(idx_i, idx_j, pp1, basis)
    ii1 = _edge_mlps(inter,
                     pi_W1, b(pi_b1), pi_W2, b(pi_b2),
                     ii_W1, b(ii_b1), ii_W2, b(ii_b2))
    zeros = jnp.zeros((N, D), jnp.float32)
    parts = _scatter_add(idx_i, ii1, zeros)
    return _add_partials(parts)


# speedup vs baseline: 3.4495x; 1.0855x over previous
import functools

import jax
import jax.numpy as jnp
from jax import lax
from jax.experimental import pallas as pl
from jax.experimental.pallas import tpu as pltpu
from jax.experimental.pallas import tpu_sc as plsc

N = 10000
E = 320000
D = 128

C = 128
NCHUNKS = E // C
NW = 32
DW = 16
ROWS_PER_SUB = 624
ROWS_LAST = N - 15 * ROWS_PER_SUB


def _node_mlp(x, W1, b1, W2, b2):
    BN = 2000

    def body(x_ref, w1_ref, b1_ref, w2_ref, b2_ref, o_ref):
        h = jnp.tanh(
            jnp.dot(x_ref[...], w1_ref[...], preferred_element_type=jnp.float32)
            + b1_ref[...]
        )
        o_ref[...] = (
            jnp.dot(h, w2_ref[...], preferred_element_type=jnp.float32)
            + b2_ref[...]
        )

    wspec = pl.BlockSpec((D, D), lambda i: (0, 0))
    bspec = pl.BlockSpec((1, D), lambda i: (0, 0))
    return pl.pallas_call(
        body,
        grid=(N // BN,),
        in_specs=[
            pl.BlockSpec((BN, D), lambda i: (i, 0)),
            wspec, bspec, wspec, bspec,
        ],
        out_specs=pl.BlockSpec((BN, D), lambda i: (i, 0)),
        out_shape=jax.ShapeDtypeStruct((N, D), jnp.float32),
    )(x, W1, b1, W2, b2)


def _edge_mlps(psum, basis, W1, b1, W2, b2, W3, b3, W4, b4):
    BE = 2000

    def body(p_ref, bas_ref, w1_ref, b1_ref, w2_ref, b2_ref,
             w3_ref, b3_ref, w4_ref, b4_ref, o_ref):
        inter = p_ref[...] + bas_ref[...]
        h1 = jnp.tanh(
            jnp.dot(inter, w1_ref[...], preferred_element_type=jnp.float32)
            + b1_ref[...]
        )
        p = (
            jnp.dot(h1, w2_ref[...], preferred_element_type=jnp.float32)
            + b2_ref[...]
        )
        h2 = jnp.tanh(
            jnp.dot(p, w3_ref[...], preferred_element_type=jnp.float32)
            + b3_ref[...]
        )
        o_ref[...] = (
            jnp.dot(h2, w4_ref[...], preferred_element_type=jnp.float32)
            + b4_ref[...]
        )

    wspec = pl.BlockSpec((D, D), lambda i: (0, 0))
    bspec = pl.BlockSpec((1, D), lambda i: (0, 0))
    espec = pl.BlockSpec((BE, D), lambda i: (i, 0))
    return pl.pallas_call(
        body,
        grid=(E // BE,),
        in_specs=[espec, espec, wspec, bspec, wspec, bspec, wspec, bspec,
                  wspec, bspec],
        out_specs=espec,
        out_shape=jax.ShapeDtypeStruct((E, D), jnp.float32),
    )(psum, basis, W1, b1, W2, b2, W3, b3, W4, b4)


def _add_partials(parts):
    BN = 2000

    def body(p_ref, o_ref):
        o_ref[...] = p_ref[0] + p_ref[1]

    return pl.pallas_call(
        body,
        grid=(N // BN,),
        in_specs=[pl.BlockSpec((2, BN, D), lambda i: (0, i, 0))],
        out_specs=pl.BlockSpec((BN, D), lambda i: (i, 0)),
        out_shape=jax.ShapeDtypeStruct((N, D), jnp.float32),
    )(parts)


_MESH = plsc.VectorSubcoreMesh(core_axis_name="c", subcore_axis_name="s")


@functools.partial(
    pl.kernel,
    mesh=_MESH,
    out_type=jax.ShapeDtypeStruct((E, D), jnp.float32),
    scratch_types=[
        pltpu.VMEM((C,), jnp.int32),
        pltpu.VMEM((C,), jnp.int32),
        pltpu.VMEM((C, D), jnp.float32),
        pltpu.VMEM((C, D), jnp.float32),
        pltpu.SemaphoreType.DMA,
        pltpu.SemaphoreType.DMA,
    ],
)
def _gather_sum(idx_i_hbm, idx_j_hbm, pp1_hbm, psum_hbm,
                idxi_v, idxj_v, rows_i, rows_j, sem_i, sem_j):
    c = lax.axis_index("c")
    s = lax.axis_index("s")
    w = s * 2 + c
    nchunks = 78 + (w < 4).astype(jnp.int32)

    def chunk(t, carry):
        g = w + t * NW
        base = pl.multiple_of(g * C, C)
        pltpu.sync_copy(idx_i_hbm.at[pl.ds(base, C)], idxi_v)
        pltpu.sync_copy(idx_j_hbm.at[pl.ds(base, C)], idxj_v)
        cp_i = pltpu.async_copy(pp1_hbm.at[idxi_v], rows_i, sem_i)
        cp_j = pltpu.async_copy(pp1_hbm.at[idxj_v], rows_j, sem_j)
        cp_i.wait()
        cp_j.wait()

        def row(r, carry2):
            for k in range(D // 16):
                sl = pl.ds(k * 16, 16)
                rows_i[r, sl] += rows_j[r, sl]
            return carry2

        lax.fori_loop(0, C, row, 0)
        pltpu.sync_copy(rows_i, psum_hbm.at[pl.ds(base, C)])
        return carry

    lax.fori_loop(0, nchunks, chunk, 0)


@functools.partial(
    pl.kernel,
    mesh=_MESH,
    out_type=jax.ShapeDtypeStruct((2, N, D), jnp.float32),
    scratch_types=[
        pltpu.VMEM((C,), jnp.int32),
        pltpu.VMEM((C, D), jnp.float32),
        pltpu.VMEM_SHARED((N, D), jnp.float32),
    ],
)
def _scatter_add(idx_i_hbm, ii1_hbm, zeros_hbm, out_hbm,
                 idx_v, rows_v, acc_shared):
    c = lax.axis_index("c")
    s = lax.axis_index("s")
    r0 = pl.multiple_of(s * ROWS_PER_SUB, 8)
    pltpu.sync_copy(zeros_hbm.at[pl.ds(r0, ROWS_PER_SUB)],
                    acc_shared.at[pl.ds(r0, ROWS_PER_SUB)])

    @pl.when(s == 15)
    def _():
        tail = 16 * ROWS_PER_SUB
        nt = ROWS_LAST - ROWS_PER_SUB
        pltpu.sync_copy(zeros_hbm.at[pl.ds(tail, nt)],
                        acc_shared.at[pl.ds(tail, nt)])

    plsc.subcore_barrier()

    half = NCHUNKS // 2
    nchunks = 78 + (s < 2).astype(jnp.int32)

    def chunk(t, carry):
        g = c * half + s + t * 16
        base = pl.multiple_of(g * C, C)
        pltpu.sync_copy(idx_i_hbm.at[pl.ds(base, C)], idx_v)
        pltpu.sync_copy(ii1_hbm.at[pl.ds(base, C)], rows_v)
        pltpu.sync_copy(rows_v, acc_shared.at[idx_v], add=True)
        return carry

    lax.fori_loop(0, nchunks, chunk, 0)
    plsc.subcore_barrier()
    pltpu.sync_copy(acc_shared.at[pl.ds(r0, ROWS_PER_SUB)],
                    out_hbm.at[c, pl.ds(r0, ROWS_PER_SUB)])

    @pl.when(s == 15)
    def _():
        tail = 16 * ROWS_PER_SUB
        nt = ROWS_LAST - ROWS_PER_SUB
        pltpu.sync_copy(acc_shared.at[pl.ds(tail, nt)],
                        out_hbm.at[c, pl.ds(tail, nt)])


def kernel(idx_i, idx_j, p1, basis,
           pp_W1, pp_b1, pp_W2, pp_b2,
           pi_W1, pi_b1, pi_W2, pi_b2,
           ii_W1, ii_b1, ii_W2, ii_b2):
    idx_i = idx_i.astype(jnp.int32)
    idx_j = idx_j.astype(jnp.int32)
    b = lambda v: v.reshape(1, D)

    pp1 = _node_mlp(p1, pp_W1, b(pp_b1), pp_W2, b(pp_b2))
    psum = _gather_sum(idx_i, idx_j, pp1)
    ii1 = _edge_mlps(psum, basis,
                     pi_W1, b(pi_b1), pi_W2, b(pi_b2),
                     ii_W1, b(ii_b1), ii_W2, b(ii_b2))
    zeros = jnp.zeros((N, D), jnp.float32)
    parts = _scatter_add(idx_i, ii1, zeros)
    return _add_partials(parts)


# speedup vs baseline: 4.4603x; 1.2930x over previous
import functools

import jax
import jax.numpy as jnp
from jax import lax
from jax.experimental import pallas as pl
from jax.experimental.pallas import tpu as pltpu
from jax.experimental.pallas import tpu_sc as plsc

N = 10000
E = 320000
D = 128

C = 128
NCHUNKS = E // C
NW = 32
DW = 16
ROWS_PER_SUB = 624
ROWS_LAST = N - 15 * ROWS_PER_SUB


def _node_mlp(x, W1, b1, W2, b2):
    BN = 2000

    def body(x_ref, w1_ref, b1_ref, w2_ref, b2_ref, o_ref):
        h = jnp.tanh(
            jnp.dot(x_ref[...], w1_ref[...], preferred_element_type=jnp.float32)
            + b1_ref[...]
        )
        o_ref[...] = (
            jnp.dot(h, w2_ref[...], preferred_element_type=jnp.float32)
            + b2_ref[...]
        )

    wspec = pl.BlockSpec((D, D), lambda i: (0, 0))
    bspec = pl.BlockSpec((1, D), lambda i: (0, 0))
    return pl.pallas_call(
        body,
        grid=(N // BN,),
        in_specs=[
            pl.BlockSpec((BN, D), lambda i: (i, 0)),
            wspec, bspec, wspec, bspec,
        ],
        out_specs=pl.BlockSpec((BN, D), lambda i: (i, 0)),
        out_shape=jax.ShapeDtypeStruct((N, D), jnp.float32),
    )(x, W1, b1, W2, b2)


def _edge_mlps(psum, basis, W1, b1, W2, b2, W3, b3, W4, b4):
    BE = 2000

    def body(p_ref, bas_ref, w1_ref, b1_ref, w2_ref, b2_ref,
             w3_ref, b3_ref, w4_ref, b4_ref, o_ref):
        inter = p_ref[...] + bas_ref[...]
        h1 = jnp.tanh(
            jnp.dot(inter, w1_ref[...], preferred_element_type=jnp.float32)
            + b1_ref[...]
        )
        p = (
            jnp.dot(h1, w2_ref[...], preferred_element_type=jnp.float32)
            + b2_ref[...]
        )
        h2 = jnp.tanh(
            jnp.dot(p, w3_ref[...], preferred_element_type=jnp.float32)
            + b3_ref[...]
        )
        o_ref[...] = (
            jnp.dot(h2, w4_ref[...], preferred_element_type=jnp.float32)
            + b4_ref[...]
        )

    wspec = pl.BlockSpec((D, D), lambda i: (0, 0))
    bspec = pl.BlockSpec((1, D), lambda i: (0, 0))
    espec = pl.BlockSpec((BE, D), lambda i: (i, 0))
    return pl.pallas_call(
        body,
        grid=(E // BE,),
        in_specs=[espec, espec, wspec, bspec, wspec, bspec, wspec, bspec,
                  wspec, bspec],
        out_specs=espec,
        out_shape=jax.ShapeDtypeStruct((E, D), jnp.float32),
    )(psum, basis, W1, b1, W2, b2, W3, b3, W4, b4)


def _add_partials(parts):
    BN = 2000

    def body(p_ref, o_ref):
        o_ref[...] = p_ref[0] + p_ref[1]

    return pl.pallas_call(
        body,
        grid=(N // BN,),
        in_specs=[pl.BlockSpec((2, BN, D), lambda i: (0, i, 0))],
        out_specs=pl.BlockSpec((BN, D), lambda i: (i, 0)),
        out_shape=jax.ShapeDtypeStruct((N, D), jnp.float32),
    )(parts)


_MESH = plsc.VectorSubcoreMesh(core_axis_name="c", subcore_axis_name="s")


CW = D
IDX_BULK = 78 * C
IDX_ALL = 79 * C


@functools.partial(
    pl.kernel,
    mesh=_MESH,
    out_type=jax.ShapeDtypeStruct((E, CW), jnp.float32),
    scratch_types=[
        pltpu.VMEM((IDX_ALL,), jnp.int32),
        pltpu.VMEM((IDX_ALL,), jnp.int32),
        pltpu.VMEM((C, CW), jnp.float32),
        pltpu.VMEM((C, CW), jnp.float32),
        pltpu.VMEM((C, CW), jnp.float32),
        pltpu.VMEM((C, CW), jnp.float32),
        pltpu.VMEM((C, CW), jnp.float32),
        pltpu.VMEM((C, CW), jnp.float32),
        pltpu.SemaphoreType.DMA,
        pltpu.SemaphoreType.DMA,
        pltpu.SemaphoreType.DMA,
        pltpu.SemaphoreType.DMA,
        pltpu.SemaphoreType.DMA,
        pltpu.SemaphoreType.DMA,
    ],
)
def _gather_sum(idx_i_hbm, idx_j_hbm, ppb_hbm, psum_hbm,
                idxi_a, idxj_a, ri0, rj0, ri1, rj1, acc0, acc1,
                gi0, gj0, gi1, gj1, wb0, wb1):
    c = lax.axis_index("c")
    s = lax.axis_index("s")
    w = s * 2 + c
    extra = w < 4
    start = 78 * w + jnp.minimum(w, 4)
    ebase = pl.multiple_of(start * C, C)

    pltpu.sync_copy(idx_i_hbm.at[pl.ds(ebase, IDX_BULK)],
                    idxi_a.at[pl.ds(0, IDX_BULK)])
    pltpu.sync_copy(idx_j_hbm.at[pl.ds(ebase, IDX_BULK)],
                    idxj_a.at[pl.ds(0, IDX_BULK)])

    @pl.when(extra)
    def _():
        pltpu.sync_copy(idx_i_hbm.at[pl.ds(ebase + IDX_BULK, C)],
                        idxi_a.at[pl.ds(IDX_BULK, C)])
        pltpu.sync_copy(idx_j_hbm.at[pl.ds(ebase + IDX_BULK, C)],
                        idxj_a.at[pl.ds(IDX_BULK, C)])

    def fire(t, ri, gi, rj, gj):
        off = t * C
        pltpu.async_copy(ppb_hbm.at[idxi_a.at[pl.ds(off, C)]], ri, gi)
        pltpu.async_copy(ppb_hbm.at[idxj_a.at[pl.ds(off, C)]], rj, gj)

    def wait_rows(ri, gi, rj, gj):
        pltpu.make_async_copy(ppb_hbm.at[pl.ds(0, C)], ri, gi).wait()
        pltpu.make_async_copy(ppb_hbm.at[pl.ds(0, C)], rj, gj).wait()

    def drain_wb(acc, wb):
        pltpu.make_async_copy(psum_hbm.at[pl.ds(0, C)], acc, wb).wait()

    def add_rows(ri, rj, acc):
        def row(r, carry):
            for k in range(CW // 16):
                sl = pl.ds(k * 16, 16)
                acc[r, sl] = ri[r, sl] + rj[r, sl]
            return carry

        lax.fori_loop(0, C, row, 0)

    def writeback(t, acc, wb):
        base = pl.multiple_of((start + t) * C, C)
        pltpu.async_copy(acc, psum_hbm.at[pl.ds(base, C)], wb)

    fire(0, ri0, gi0, rj0, gj0)
    fire(1, ri1, gi1, rj1, gj1)

    def pair(i, carry):
        t0 = 2 * i
        wait_rows(ri0, gi0, rj0, gj0)

        @pl.when(i > 0)
        def _():
            drain_wb(acc0, wb0)

        add_rows(ri0, rj0, acc0)
        writeback(t0, acc0, wb0)

        @pl.when(i < 38)
        def _():
            fire(t0 + 2, ri0, gi0, rj0, gj0)

        wait_rows(ri1, gi1, rj1, gj1)

        @pl.when(i > 0)
        def _():
            drain_wb(acc1, wb1)

        add_rows(ri1, rj1, acc1)
        writeback(t0 + 1, acc1, wb1)

        @pl.when(i < 38)
        def _():
            fire(t0 + 3, ri1, gi1, rj1, gj1)

        return carry

    lax.fori_loop(0, 39, pair, 0)

    @pl.when(extra)
    def _():
        fire(78, ri0, gi0, rj0, gj0)
        wait_rows(ri0, gi0, rj0, gj0)
        drain_wb(acc0, wb0)
        add_rows(ri0, rj0, acc0)
        base = pl.multiple_of((start + 78) * C, C)
        pltpu.sync_copy(acc0, psum_hbm.at[pl.ds(base, C)])

    @pl.when(jnp.logical_not(extra))
    def _():
        drain_wb(acc0, wb0)

    drain_wb(acc1, wb1)


@functools.partial(
    pl.kernel,
    mesh=_MESH,
    out_type=jax.ShapeDtypeStruct((2, N, D), jnp.float32),
    scratch_types=[
        pltpu.VMEM((C,), jnp.int32),
        pltpu.VMEM((C, D), jnp.float32),
        pltpu.VMEM_SHARED((N, D), jnp.float32),
    ],
)
def _scatter_add(idx_i_hbm, ii1_hbm, zeros_hbm, out_hbm,
                 idx_v, rows_v, acc_shared):
    c = lax.axis_index("c")
    s = lax.axis_index("s")
    r0 = pl.multiple_of(s * ROWS_PER_SUB, 8)
    pltpu.sync_copy(zeros_hbm.at[pl.ds(r0, ROWS_PER_SUB)],
                    acc_shared.at[pl.ds(r0, ROWS_PER_SUB)])

    @pl.when(s == 15)
    def _():
        tail = 16 * ROWS_PER_SUB
        nt = ROWS_LAST - ROWS_PER_SUB
        pltpu.sync_copy(zeros_hbm.at[pl.ds(tail, nt)],
                        acc_shared.at[pl.ds(tail, nt)])

    plsc.subcore_barrier()

    half = NCHUNKS // 2
    nchunks = 78 + (s < 2).astype(jnp.int32)

    def chunk(t, carry):
        g = c * half + s + t * 16
        base = pl.multiple_of(g * C, C)
        pltpu.sync_copy(idx_i_hbm.at[pl.ds(base, C)], idx_v)
        pltpu.sync_copy(ii1_hbm.at[pl.ds(base, C)], rows_v)
        pltpu.sync_copy(rows_v, acc_shared.at[idx_v], add=True)
        return carry

    lax.fori_loop(0, nchunks, chunk, 0)
    plsc.subcore_barrier()
    pltpu.sync_copy(acc_shared.at[pl.ds(r0, ROWS_PER_SUB)],
                    out_hbm.at[c, pl.ds(r0, ROWS_PER_SUB)])

    @pl.when(s == 15)
    def _():
        tail = 16 * ROWS_PER_SUB
        nt = ROWS_LAST - ROWS_PER_SUB
        pltpu.sync_copy(acc_shared.at[pl.ds(tail, nt)],
                        out_hbm.at[c, pl.ds(tail, nt)])


def kernel(idx_i, idx_j, p1, basis,
           pp_W1, pp_b1, pp_W2, pp_b2,
           pi_W1, pi_b1, pi_W2, pi_b2,
           ii_W1, ii_b1, ii_W2, ii_b2):
    idx_i = idx_i.astype(jnp.int32)
    idx_j = idx_j.astype(jnp.int32)
    b = lambda v: v.reshape(1, D)

    pp1 = _node_mlp(p1, pp_W1, b(pp_b1), pp_W2, b(pp_b2))
    psum = _gather_sum(idx_i, idx_j, pp1)
    ii1 = _edge_mlps(psum, basis,
                     pi_W1, b(pi_b1), pi_W2, b(pi_b2),
                     ii_W1, b(ii_b1), ii_W2, b(ii_b2))
    zeros = jnp.zeros((N, D), jnp.float32)
    parts = _scatter_add(idx_i, ii1, zeros)
    return _add_partials(parts)


# speedup vs baseline: 5.2606x; 1.1794x over previous
import functools

import jax
import jax.numpy as jnp
from jax import lax
from jax.experimental import pallas as pl
from jax.experimental.pallas import tpu as pltpu
from jax.experimental.pallas import tpu_sc as plsc

N = 10000
E = 320000
D = 128

C = 128
NCHUNKS = E // C
NW = 32
DW = 16
ROWS_PER_SUB = 624
ROWS_LAST = N - 15 * ROWS_PER_SUB


def _node_mlp(x, W1, b1, W2, b2):
    BN = 2000

    def body(x_ref, w1_ref, b1_ref, w2_ref, b2_ref, o_ref):
        h = jnp.tanh(
            jnp.dot(x_ref[...], w1_ref[...], preferred_element_type=jnp.float32)
            + b1_ref[...]
        )
        o_ref[...] = (
            jnp.dot(h, w2_ref[...], preferred_element_type=jnp.float32)
            + b2_ref[...]
        )

    wspec = pl.BlockSpec((D, D), lambda i: (0, 0))
    bspec = pl.BlockSpec((1, D), lambda i: (0, 0))
    return pl.pallas_call(
        body,
        grid=(N // BN,),
        in_specs=[
            pl.BlockSpec((BN, D), lambda i: (i, 0)),
            wspec, bspec, wspec, bspec,
        ],
        out_specs=pl.BlockSpec((BN, D), lambda i: (i, 0)),
        out_shape=jax.ShapeDtypeStruct((N, D), jnp.float32),
    )(x, W1, b1, W2, b2)


def _edge_mlps(psum, basis, W1, b1, W2, b2, W3, b3, W4, b4):
    BE = 2000

    def body(p_ref, bas_ref, w1_ref, b1_ref, w2_ref, b2_ref,
             w3_ref, b3_ref, w4_ref, b4_ref, o_ref):
        inter = p_ref[...] + bas_ref[...]
        h1 = jnp.tanh(
            jnp.dot(inter, w1_ref[...], preferred_element_type=jnp.float32)
            + b1_ref[...]
        )
        p = (
            jnp.dot(h1, w2_ref[...], preferred_element_type=jnp.float32)
            + b2_ref[...]
        )
        h2 = jnp.tanh(
            jnp.dot(p, w3_ref[...], preferred_element_type=jnp.float32)
            + b3_ref[...]
        )
        o_ref[...] = (
            jnp.dot(h2, w4_ref[...], preferred_element_type=jnp.float32)
            + b4_ref[...]
        )

    wspec = pl.BlockSpec((D, D), lambda i: (0, 0))
    bspec = pl.BlockSpec((1, D), lambda i: (0, 0))
    espec = pl.BlockSpec((BE, D), lambda i: (i, 0))
    return pl.pallas_call(
        body,
        grid=(E // BE,),
        in_specs=[espec, espec, wspec, bspec, wspec, bspec, wspec, bspec,
                  wspec, bspec],
        out_specs=espec,
        out_shape=jax.ShapeDtypeStruct((E, D), jnp.float32),
    )(psum, basis, W1, b1, W2, b2, W3, b3, W4, b4)


def _add_partials(parts):
    BN = 2000

    def body(p_ref, o_ref):
        o_ref[...] = p_ref[0] + p_ref[1]

    return pl.pallas_call(
        body,
        grid=(N // BN,),
        in_specs=[pl.BlockSpec((2, BN, D), lambda i: (0, i, 0))],
        out_specs=pl.BlockSpec((BN, D), lambda i: (i, 0)),
        out_shape=jax.ShapeDtypeStruct((N, D), jnp.float32),
    )(parts)


_MESH = plsc.VectorSubcoreMesh(core_axis_name="c", subcore_axis_name="s")


CW = D
IDX_BULK = 78 * C
IDX_ALL = 79 * C


@functools.partial(
    pl.kernel,
    mesh=_MESH,
    out_type=jax.ShapeDtypeStruct((E, CW), jnp.float32),
    scratch_types=[
        pltpu.VMEM((IDX_ALL,), jnp.int32),
        pltpu.VMEM((IDX_ALL,), jnp.int32),
        pltpu.VMEM((C, CW), jnp.float32),
        pltpu.VMEM((C, CW), jnp.float32),
        pltpu.VMEM((C, CW), jnp.float32),
        pltpu.VMEM((C, CW), jnp.float32),
        pltpu.VMEM((C, CW), jnp.float32),
        pltpu.VMEM((C, CW), jnp.float32),
        pltpu.SemaphoreType.DMA,
        pltpu.SemaphoreType.DMA,
        pltpu.SemaphoreType.DMA,
        pltpu.SemaphoreType.DMA,
        pltpu.SemaphoreType.DMA,
        pltpu.SemaphoreType.DMA,
    ],
)
def _gather_sum(idx_i_hbm, idx_j_hbm, ppb_hbm, psum_hbm,
                idxi_a, idxj_a, ri0, rj0, ri1, rj1, acc0, acc1,
                gi0, gj0, gi1, gj1, wb0, wb1):
    c = lax.axis_index("c")
    s = lax.axis_index("s")
    w = s * 2 + c
    extra = w < 4
    start = 78 * w + jnp.minimum(w, 4)
    ebase = pl.multiple_of(start * C, C)

    pltpu.sync_copy(idx_i_hbm.at[pl.ds(ebase, IDX_BULK)],
                    idxi_a.at[pl.ds(0, IDX_BULK)])
    pltpu.sync_copy(idx_j_hbm.at[pl.ds(ebase, IDX_BULK)],
                    idxj_a.at[pl.ds(0, IDX_BULK)])

    @pl.when(extra)
    def _():
        pltpu.sync_copy(idx_i_hbm.at[pl.ds(ebase + IDX_BULK, C)],
                        idxi_a.at[pl.ds(IDX_BULK, C)])
        pltpu.sync_copy(idx_j_hbm.at[pl.ds(ebase + IDX_BULK, C)],
                        idxj_a.at[pl.ds(IDX_BULK, C)])

    def fire(t, ri, gi, rj, gj):
        off = t * C
        pltpu.async_copy(ppb_hbm.at[idxi_a.at[pl.ds(off, C)]], ri, gi)
        pltpu.async_copy(ppb_hbm.at[idxj_a.at[pl.ds(off, C)]], rj, gj)

    def wait_rows(ri, gi, rj, gj):
        pltpu.make_async_copy(ppb_hbm.at[pl.ds(0, C)], ri, gi).wait()
        pltpu.make_async_copy(ppb_hbm.at[pl.ds(0, C)], rj, gj).wait()

    def drain_wb(acc, wb):
        pltpu.make_async_copy(psum_hbm.at[pl.ds(0, C)], acc, wb).wait()

    def add_rows(ri, rj, acc):
        def row(r, carry):
            for k in range(CW // 16):
                sl = pl.ds(k * 16, 16)
                acc[r, sl] = ri[r, sl] + rj[r, sl]
            return carry

        lax.fori_loop(0, C, row, 0)

    def writeback(t, acc, wb):
        base = pl.multiple_of((start + t) * C, C)
        pltpu.async_copy(acc, psum_hbm.at[pl.ds(base, C)], wb)

    fire(0, ri0, gi0, rj0, gj0)
    fire(1, ri1, gi1, rj1, gj1)

    def pair(i, carry):
        t0 = 2 * i
        wait_rows(ri0, gi0, rj0, gj0)

        @pl.when(i > 0)
        def _():
            drain_wb(acc0, wb0)

        add_rows(ri0, rj0, acc0)
        writeback(t0, acc0, wb0)

        @pl.when(i < 38)
        def _():
            fire(t0 + 2, ri0, gi0, rj0, gj0)

        wait_rows(ri1, gi1, rj1, gj1)

        @pl.when(i > 0)
        def _():
            drain_wb(acc1, wb1)

        add_rows(ri1, rj1, acc1)
        writeback(t0 + 1, acc1, wb1)

        @pl.when(i < 38)
        def _():
            fire(t0 + 3, ri1, gi1, rj1, gj1)

        return carry

    lax.fori_loop(0, 39, pair, 0)

    @pl.when(extra)
    def _():
        fire(78, ri0, gi0, rj0, gj0)
        wait_rows(ri0, gi0, rj0, gj0)
        drain_wb(acc0, wb0)
        add_rows(ri0, rj0, acc0)
        base = pl.multiple_of((start + 78) * C, C)
        pltpu.sync_copy(acc0, psum_hbm.at[pl.ds(base, C)])

    @pl.when(jnp.logical_not(extra))
    def _():
        drain_wb(acc0, wb0)

    drain_wb(acc1, wb1)


@functools.partial(
    pl.kernel,
    mesh=_MESH,
    out_type=jax.ShapeDtypeStruct((2, N, D), jnp.float32),
    scratch_types=[
        pltpu.VMEM((79, C), jnp.int32),
        pltpu.VMEM((C, D), jnp.float32),
        pltpu.VMEM((C, D), jnp.float32),
        pltpu.VMEM_SHARED((N, D), jnp.float32),
        pltpu.SemaphoreType.DMA,
        pltpu.SemaphoreType.DMA,
        pltpu.SemaphoreType.DMA,
    ],
)
def _scatter_add(idx_i_hbm, ii1_hbm, zeros_hbm, out_hbm,
                 idx2d, rows0, rows1, acc_shared, six, sr0, sr1):
    c = lax.axis_index("c")
    s = lax.axis_index("s")
    w = s * 2 + c
    extra = w < 4
    start = 78 * w + jnp.minimum(w, 4)

    def fire_idx(t, carry):
        base = pl.multiple_of((start + t) * C, C)
        pltpu.async_copy(idx_i_hbm.at[pl.ds(base, C)], idx2d.at[t], six)
        return carry

    lax.fori_loop(0, 78, fire_idx, 0)

    @pl.when(extra)
    def _():
        base = pl.multiple_of((start + 78) * C, C)
        pltpu.async_copy(idx_i_hbm.at[pl.ds(base, C)], idx2d.at[78], six)

    r0 = pl.multiple_of(s * ROWS_PER_SUB, 8)
    pltpu.sync_copy(zeros_hbm.at[pl.ds(r0, ROWS_PER_SUB)],
                    acc_shared.at[pl.ds(r0, ROWS_PER_SUB)])

    @pl.when(s == 15)
    def _():
        tail = 16 * ROWS_PER_SUB
        nt = ROWS_LAST - ROWS_PER_SUB
        pltpu.sync_copy(zeros_hbm.at[pl.ds(tail, nt)],
                        acc_shared.at[pl.ds(tail, nt)])

    def drain_idx(t, carry):
        pltpu.make_async_copy(idx_i_hbm.at[pl.ds(0, C)], idx2d.at[t], six
                              ).wait()
        return carry

    lax.fori_loop(0, 78, drain_idx, 0)

    @pl.when(extra)
    def _():
        pltpu.make_async_copy(idx_i_hbm.at[pl.ds(0, C)], idx2d.at[78], six
                              ).wait()

    plsc.subcore_barrier()

    def fire_rows(t, rows, sem):
        base = pl.multiple_of((start + t) * C, C)
        pltpu.async_copy(ii1_hbm.at[pl.ds(base, C)], rows, sem)

    def wait_rows(rows, sem):
        pltpu.make_async_copy(ii1_hbm.at[pl.ds(0, C)], rows, sem).wait()

    fire_rows(0, rows0, sr0)
    fire_rows(1, rows1, sr1)

    def pair(i, carry):
        t0 = 2 * i
        wait_rows(rows0, sr0)
        pltpu.sync_copy(rows0, acc_shared.at[idx2d.at[t0]], add=True)

        @pl.when(i < 38)
        def _():
            fire_rows(t0 + 2, rows0, sr0)

        wait_rows(rows1, sr1)
        pltpu.sync_copy(rows1, acc_shared.at[idx2d.at[t0 + 1]], add=True)

        @pl.when(i < 38)
        def _():
            fire_rows(t0 + 3, rows1, sr1)

        return carry

    lax.fori_loop(0, 39, pair, 0)

    @pl.when(extra)
    def _():
        fire_rows(78, rows0, sr0)
        wait_rows(rows0, sr0)
        pltpu.sync_copy(rows0, acc_shared.at[idx2d.at[78]], add=True)

    plsc.subcore_barrier()
    pltpu.sync_copy(acc_shared.at[pl.ds(r0, ROWS_PER_SUB)],
                    out_hbm.at[c, pl.ds(r0, ROWS_PER_SUB)])

    @pl.when(s == 15)
    def _():
        tail = 16 * ROWS_PER_SUB
        nt = ROWS_LAST - ROWS_PER_SUB
        pltpu.sync_copy(acc_shared.at[pl.ds(tail, nt)],
                        out_hbm.at[c, pl.ds(tail, nt)])


def kernel(idx_i, idx_j, p1, basis,
           pp_W1, pp_b1, pp_W2, pp_b2,
           pi_W1, pi_b1, pi_W2, pi_b2,
           ii_W1, ii_b1, ii_W2, ii_b2):
    idx_i = idx_i.astype(jnp.int32)
    idx_j = idx_j.astype(jnp.int32)
    b = lambda v: v.reshape(1, D)

    pp1 = _node_mlp(p1, pp_W1, b(pp_b1), pp_W2, b(pp_b2))
    psum = _gather_sum(idx_i, idx_j, pp1)
    ii1 = _edge_mlps(psum, basis,
                     pi_W1, b(pi_b1), pi_W2, b(pi_b2),
                     ii_W1, b(ii_b1), ii_W2, b(ii_b2))
    zeros = jnp.zeros((N, D), jnp.float32)
    parts = _scatter_add(idx_i, ii1, zeros)
    return _add_partials(parts)
